# baseline (speedup 1.0000x reference)
import functools

import jax
import jax.numpy as jnp
from jax import lax
from jax.experimental import pallas as pl
from jax.experimental.pallas import tpu as pltpu
from jax.experimental.pallas import tpu_sc as plsc

_BSZ, _SEQ, _DIM = 16, 2048, 256
_NDOM, _NSLOT = 8, 16
_L = 16
_GROUPS = _SEQ // _L
_CH = 256
_NCH = _SEQ // _CH
_GPC = _CH // _L
_GUARD = _SEQ // 2
_WIN = 1024
_NPASS = _SEQ // _WIN
_WROWS = _WIN + 1
_WPS = 8


def _tc_scores_body(dom_ref, h_ref, emb_ref, out_ref):
    d = dom_ref[pl.program_id(0)]
    e = emb_ref[d]
    h = h_ref[0]
    out_ref[0] = jax.lax.dot_general(
        h, e, (((1,), (1,)), ((), ())),
        preferred_element_type=jnp.float32)


def _tc_scores(domains, hidden, slot_embs):
    grid_spec = pltpu.PrefetchScalarGridSpec(
        num_scalar_prefetch=1,
        grid=(_BSZ,),
        in_specs=[
            pl.BlockSpec((1, _SEQ, _DIM), lambda b, dom: (b, 0, 0)),
            pl.BlockSpec((_NDOM, _NSLOT, _DIM), lambda b, dom: (0, 0, 0)),
        ],
        out_specs=pl.BlockSpec((1, _SEQ, _NSLOT), lambda b, dom: (b, 0, 0)),
    )
    return pl.pallas_call(
        _tc_scores_body,
        grid_spec=grid_spec,
        out_shape=jax.ShapeDtypeStruct((_BSZ, _SEQ, _NSLOT), jnp.float32),
    )(domains, hidden, slot_embs)


def _sc_segsum_body(scores_hbm, labels_hbm, out_hbm,
                    lab_v, sbufa_v, sbufb_v,
                    idx0, idx1, idx2, idx3, idx4, idx5, idx6, idx7,
                    src_v, acc_sh, sem):
    idx_bufs = (idx0, idx1, idx2, idx3, idx4, idx5, idx6, idx7)
    core = lax.axis_index("c")
    sub = lax.axis_index("s")

    @pl.when(sub < _WPS)
    def _():
        b = core * _WPS + sub
        base = sub * _WROWS

        cl = pltpu.async_copy(labels_hbm.at[b], lab_v, sem)

        def _zero(i, c):
            src_v[i] = jnp.zeros((_L,), jnp.float32)
            return c
        lax.fori_loop(0, _CH, _zero, 0, unroll=8)

        def _zg(i, c):
            sbufa_v[pl.ds(i * _L, _L)] = jnp.zeros((_L,), jnp.int32)
            sbufb_v[pl.ds(i * _L, _L)] = jnp.zeros((_L,), jnp.int32)
            return c
        lax.fori_loop(0, _GUARD // _L, _zg, 0, unroll=8)

        cl.wait()

        def _init(g, c):
            lab = lab_v[pl.ds(g * _L, _L)]
            sbufa_v[pl.ds(_GUARD + g * _L, _L)] = jnp.where(
                lab == 1, jnp.full((_L,), 1, jnp.int32),
                jnp.full((_L,), 0, jnp.int32))
            return c
        lax.fori_loop(0, _GROUPS, _init, 0, unroll=4)

        bufs = (sbufa_v, sbufb_v)
        shift = 1
        for p in range(11):
            src, dst = bufs[p % 2], bufs[(p + 1) % 2]
            k = shift

            def _scan(g, c, src=src, dst=dst, k=k):
                x = src[pl.ds(_GUARD + g * _L, _L)]
                y = src[pl.ds(_GUARD + g * _L - k, _L)]
                dst[pl.ds(_GUARD + g * _L, _L)] = x + y
                return c
            lax.fori_loop(0, _GROUPS, _scan, 0, unroll=4)
            shift *= 2
        cum_v = bufs[11 % 2]
        glob_v = bufs[0]

        def _tgt(g, c):
            lab = lab_v[pl.ds(g * _L, _L)]
            cum = cum_v[pl.ds(_GUARD + g * _L, _L)]
            kept = (lab == 1) | (lab == 2)
            glob_v[pl.ds(_GUARD + g * _L, _L)] = jnp.where(
                kept, cum - 1, jnp.full((_L,), _SEQ, jnp.int32))
            return c
        lax.fori_loop(0, _GROUPS, _tgt, 0, unroll=4)

        for ps in range(_NPASS):
            lo = ps * _WIN

            for c in range(_WIN // _CH):
                pltpu.sync_copy(src_v, acc_sh.at[pl.ds(base + c * _CH, _CH)])

            for ch in range(_NCH):
                ib = idx_bufs[ch]

                def _loc(j, c2, ch=ch, ib=ib, lo=lo):
                    g = ch * _GPC + j
                    row = glob_v[pl.ds(_GUARD + g * _L, _L)] - lo
                    inwin = (row >= 0) & (row < _WIN)
                    ib[pl.ds(j * _L, _L)] = base + jnp.where(
                        inwin, row, jnp.full((_L,), _WIN, jnp.int32))
                    return c2
                lax.fori_loop(0, _GPC, _loc, 0, unroll=4)

            for ch in range(_NCH):
                pltpu.sync_copy(scores_hbm.at[b, pl.ds(ch * _CH, _CH)], src_v)
                pltpu.sync_copy(src_v, acc_sh.at[idx_bufs[ch]], add=True)

            pltpu.sync_copy(acc_sh.at[pl.ds(base, _WIN)],
                            out_hbm.at[b, pl.ds(lo, _WIN)])

            if ps + 1 < _NPASS:
                def _rz(i, c2):
                    src_v[i] = jnp.zeros((_L,), jnp.float32)
                    return c2
                lax.fori_loop(0, _CH, _rz, 0, unroll=8)


_sc_segsum = functools.partial(
    pl.kernel,
    mesh=plsc.VectorSubcoreMesh(core_axis_name="c", subcore_axis_name="s"),
    out_type=jax.ShapeDtypeStruct((_BSZ, _SEQ, _NSLOT), jnp.float32),
    scratch_types=[
        pltpu.VMEM((_SEQ,), jnp.int32),
        pltpu.VMEM((_GUARD + _SEQ,), jnp.int32),
        pltpu.VMEM((_GUARD + _SEQ,), jnp.int32),
        *[pltpu.VMEM((_CH,), jnp.int32) for _ in range(_NCH)],
        pltpu.VMEM((_CH, _NSLOT), jnp.float32),
        pltpu.VMEM_SHARED((_WPS * _WROWS, _NSLOT), jnp.float32),
        pltpu.SemaphoreType.DMA,
    ],
)(_sc_segsum_body)


def kernel(hidden_layers, slot_embs, domains, binary_golds):
    scores = _tc_scores(domains.astype(jnp.int32), hidden_layers, slot_embs)
    return _sc_segsum(scores, binary_golds.astype(jnp.int32))
